# Initial kernel scaffold; baseline (speedup 1.0000x reference)
import jax
import jax.numpy as jnp
from jax.experimental import pallas as pl


def kernel(x, local_angles, ent_angles):
    raise NotImplementedError("write your pallas kernel here")


# speedup vs baseline: 7.7828x; 7.7828x over previous
import jax
import jax.numpy as jnp
import numpy as np
from jax.experimental import pallas as pl

_D = 1024
_NL = 2
_NSTEP = 3 * _NL


def _tc_body(theta_ref, x_ref, o_ref):
    v = x_ref[...]
    lane = jax.lax.broadcasted_iota(jnp.int32, (1, _D), 1)
    parity = lane % 2
    for k in range(_NSTEP):
        p = 1 if (k % 3 == 1) else 0
        t = theta_ref[k, :].reshape(1, _D)
        c = jnp.cos(t)
        s = jnp.sin(t)
        is_lo = parity == p
        s_signed = jnp.where(is_lo, s, -s)
        partner = jnp.where(is_lo, jnp.roll(v, -1, axis=1), jnp.roll(v, 1, axis=1))
        v = c * v + s_signed * partner
    o_ref[...] = v


def _theta_table(local_angles, ent_angles):
    evenm = jnp.asarray((np.arange(_D) % 2) == 0)
    rows = []
    for l in range(_NL):
        a = local_angles[l]
        rows.append(jnp.where(evenm, a, jnp.roll(a, 1)))
        rows.append(jnp.where(~evenm, a, jnp.roll(a, 1)))
        rows.append(jnp.repeat(ent_angles[l], 2))
    return jnp.stack(rows)


def kernel(x, local_angles, ent_angles):
    b, s, d = x.shape
    xf = x.reshape(b * s, d)
    theta = _theta_table(local_angles, ent_angles)
    rows = b * s
    br = 512
    out = pl.pallas_call(
        _tc_body,
        grid=(rows // br,),
        in_specs=[
            pl.BlockSpec((_NSTEP, d), lambda i: (0, 0)),
            pl.BlockSpec((br, d), lambda i: (i, 0)),
        ],
        out_specs=pl.BlockSpec((br, d), lambda i: (i, 0)),
        out_shape=jax.ShapeDtypeStruct((rows, d), x.dtype),
    )(theta, xf)
    return out.reshape(b, s, d)
